# Initial kernel scaffold; baseline (speedup 1.0000x reference)
import jax
import jax.numpy as jnp
from jax.experimental import pallas as pl


def kernel(inputs, W_enc0, b_enc0, W_enc1, b_enc1, codebook0, codebook1, W_dec0, b_dec0, W_dec1_up, b_dec1_up, W_dec1_out, b_dec1_out, commitment):
    raise NotImplementedError("write your pallas kernel here")


# speedup vs baseline: 1.6227x; 1.6227x over previous
import functools

import jax
import jax.numpy as jnp
from jax import lax
from jax.experimental import pallas as pl
from jax.experimental.pallas import tpu as pltpu

_M1 = 6272
_BM1 = 392
_GRID = _M1 // _BM1
_K = 512
_D = 128
_N0 = 4 * _M1
_N1PAD = 6400
_NW = 32


def _enc_body(x0_ref, we0_ref, be0_ref, we1_ref, be1_ref, cb0t_ref, cb1t_ref,
              enc0_ref, idx0_ref, idx1_ref, s0_ref, s1_ref, acc_sc):
    i = pl.program_id(0)

    @pl.when(i == 0)
    def _init():
        s0_ref[0, 0] = 0.0
        s1_ref[0, 0] = 0.0

    f32 = jnp.float32
    iota = jax.lax.broadcasted_iota(jnp.int32, (_BM1, _K), 1)

    def vq_idx(e, cbt_ref):
        cbt = cbt_ref[...]
        g = jax.lax.dot_general(e, cbt, (((1,), (0,)), ((), ())),
                                preferred_element_type=f32)
        sc = jnp.sum(cbt * cbt, axis=0, keepdims=True) - 2.0 * g
        m = jnp.min(sc, axis=1, keepdims=True)
        idx = jnp.min(jnp.where(sc == m, iota, _K), axis=1, keepdims=True)
        return idx, jnp.sum(e * e) + jnp.sum(m)

    acc_sc[...] = jnp.broadcast_to(be1_ref[...], (_BM1, _D))

    def enc_step(j, s0):
        e0 = jax.lax.dot_general(x0_ref[j], we0_ref[...],
                                 (((1,), (0,)), ((), ())),
                                 preferred_element_type=f32) + be0_ref[...]
        idx, ds = vq_idx(e0, cb0t_ref)
        enc0_ref[j] = e0
        idx0_ref[j] = idx
        acc_sc[...] += jax.lax.dot_general(e0, we1_ref[j],
                                           (((1,), (0,)), ((), ())),
                                           preferred_element_type=f32)
        return s0 + ds

    s0 = jax.lax.fori_loop(0, 4, enc_step, 0.0)
    idx1, ds1 = vq_idx(acc_sc[...], cb1t_ref)
    idx1_ref[...] = idx1
    s0_ref[0, 0] += s0
    s1_ref[0, 0] += ds1


def _dec_body(x0_ref, enc0_ref, emb0_ref, emb1_ref, wd0_ref, bd0_ref,
              wup_ref, bup_ref, wout_ref, bout_ref, sm0_ref, sfin_ref):
    i = pl.program_id(0)

    @pl.when(i == 0)
    def _init():
        sm0_ref[0, 0] = 0.0
        sfin_ref[0, 0] = 0.0

    f32 = jnp.float32

    def dec_step(j, carry):
        sm0, sfin = carry
        emb1 = emb1_ref[...]
        d0 = jax.lax.dot_general(emb1, wd0_ref[j], (((1,), (0,)), ((), ())),
                                 preferred_element_type=f32) + bd0_ref[j]
        d = d0 - enc0_ref[j]
        up = jax.lax.dot_general(emb1, wup_ref[j], (((1,), (0,)), ((), ())),
                                 preferred_element_type=f32) + bup_ref[j]
        h = jnp.maximum(jnp.concatenate([up, emb0_ref[j]], axis=1), 0.0)
        r = jax.lax.dot_general(h, wout_ref[...], (((1,), (0,)), ((), ())),
                                preferred_element_type=f32) + bout_ref[...]
        dr = r - x0_ref[j]
        return sm0 + jnp.sum(d * d), sfin + jnp.sum(dr * dr)

    sm0, sfin = jax.lax.fori_loop(0, 4, dec_step, (0.0, 0.0))
    sm0_ref[0, 0] += sm0
    sfin_ref[0, 0] += sfin


def _full(shape):
    return pl.BlockSpec(shape, lambda i: tuple(0 for _ in shape))


_SCALAR = pl.BlockSpec((1, 1), lambda i: (0, 0), memory_space=pltpu.SMEM)


def _enc_call(x0, we0, be0, we1, be1, cb0t, cb1t, interpret=False):
    return pl.pallas_call(
        _enc_body,
        grid=(_GRID,),
        in_specs=[
            pl.BlockSpec((4, _BM1, 48), lambda i: (0, i, 0)),
            _full((48, _D)), _full((1, _D)),
            _full((4, _D, _D)), _full((1, _D)),
            _full((_D, _K)), _full((_D, _K)),
        ],
        out_specs=[
            pl.BlockSpec((4, _BM1, _D), lambda i: (0, i, 0)),
            pl.BlockSpec((4, _BM1, 1), lambda i: (0, i, 0)),
            pl.BlockSpec((_BM1, 1), lambda i: (i, 0)),
            _SCALAR, _SCALAR,
        ],
        out_shape=[
            jax.ShapeDtypeStruct((4, _M1, _D), jnp.float32),
            jax.ShapeDtypeStruct((4, _M1, 1), jnp.int32),
            jax.ShapeDtypeStruct((_M1, 1), jnp.int32),
            jax.ShapeDtypeStruct((1, 1), jnp.float32),
            jax.ShapeDtypeStruct((1, 1), jnp.float32),
        ],
        scratch_shapes=[pltpu.VMEM((_BM1, _D), jnp.float32)],
        interpret=interpret,
    )(x0, we0, be0, we1, be1, cb0t, cb1t)


def _dec_call(x0, enc0, emb0, emb1, wd0, bd0, wup, bup, wout, bout,
              interpret=False):
    return pl.pallas_call(
        _dec_body,
        grid=(_GRID,),
        in_specs=[
            pl.BlockSpec((4, _BM1, 48), lambda i: (0, i, 0)),
            pl.BlockSpec((4, _BM1, _D), lambda i: (0, i, 0)),
            pl.BlockSpec((4, _BM1, _D), lambda i: (0, i, 0)),
            pl.BlockSpec((_BM1, _D), lambda i: (i, 0)),
            _full((4, _D, _D)), _full((4, 1, _D)),
            _full((4, _D, _D)), _full((4, 1, _D)),
            _full((2 * _D, 48)), _full((1, 48)),
        ],
        out_specs=[_SCALAR, _SCALAR],
        out_shape=[jax.ShapeDtypeStruct((1, 1), jnp.float32)] * 2,
        interpret=interpret,
    )(x0, enc0, emb0, emb1, wd0, bd0, wup, bup, wout, bout)


@functools.cache
def _make_gather(B):
    from jax.experimental.pallas import tpu_sc as plsc
    b_per_w = B // _NW
    mesh = plsc.VectorSubcoreMesh(core_axis_name="c", subcore_axis_name="s")

    @functools.partial(
        pl.kernel, mesh=mesh,
        out_type=jax.ShapeDtypeStruct((B, _D), jnp.float32),
        scratch_types=[pltpu.VMEM((b_per_w,), jnp.int32),
                       pltpu.VMEM((b_per_w, _D), jnp.float32),
                       pltpu.SemaphoreType.DMA],
    )
    def gather_k(table_hbm, idx_hbm, out_hbm, idx_v, rows_v, sem):
        wid = lax.axis_index("s") * 2 + lax.axis_index("c")
        base = wid * b_per_w
        pltpu.sync_copy(idx_hbm.at[pl.ds(base, b_per_w)], idx_v)
        pltpu.async_copy(table_hbm.at[idx_v], rows_v, sem).wait()
        pltpu.sync_copy(rows_v, out_hbm.at[pl.ds(base, b_per_w)])

    return gather_k


def kernel(inputs, W_enc0, b_enc0, W_enc1, b_enc1, codebook0, codebook1,
           W_dec0, b_dec0, W_dec1_up, b_dec1_up, W_dec1_out, b_dec1_out,
           commitment=0.25):
    x0 = (inputs.reshape(8, 28, 2, 4, 28, 2, 4, 3)
          .transpose(2, 5, 0, 1, 4, 3, 6, 7)
          .reshape(4, _M1, 48))
    we1 = W_enc1.reshape(4, _D, _D)
    wd0 = W_dec0.reshape(_D, 4, _D).transpose(1, 0, 2)
    bd0 = b_dec0.reshape(4, 1, _D)
    wup = W_dec1_up.reshape(_D, 4, _D).transpose(1, 0, 2)
    bup = b_dec1_up.reshape(4, 1, _D)

    enc0, idx0, idx1, s0, s1 = _enc_call(
        x0, W_enc0, b_enc0.reshape(1, _D), we1, b_enc1.reshape(1, _D),
        codebook0.T, codebook1.T)

    emb0 = _make_gather(_N0)(codebook0, idx0.reshape(_N0))
    idx1p = jnp.pad(idx1.reshape(_M1), (0, _N1PAD - _M1))
    emb1 = _make_gather(_N1PAD)(codebook1, idx1p)

    sm0, sfin = _dec_call(
        x0, enc0, emb0.reshape(4, _M1, _D), emb1,
        wd0, bd0, wup, bup, W_dec1_out, b_dec1_out.reshape(1, 48))

    vq_loss = (1.0 + commitment) * (s0[0, 0] / (_N0 * _D)
                                    + s1[0, 0] / (_M1 * _D))
    mse0 = sm0[0, 0] / (_M1 * 4 * _D)
    final_mse = sfin[0, 0] / (_N0 * 48.0)
    loss = vq_loss + mse0 + final_mse
    return loss, final_mse


# speedup vs baseline: 1.7903x; 1.1033x over previous
import functools

import jax
import jax.numpy as jnp
from jax import lax
from jax.experimental import pallas as pl
from jax.experimental.pallas import tpu as pltpu

_M1 = 6272
_BM1 = 392
_GRID = _M1 // _BM1
_K = 512
_D = 128
_N0 = 4 * _M1
_N1PAD = 6400
_NW = 32


def _enc_body(x0_ref, we0_ref, be0_ref, we1_ref, be1_ref, cb0t_ref, cb1t_ref,
              enc0_ref, idx0_ref, idx1_ref, s0_ref, s1_ref, acc_sc):
    i = pl.program_id(0)

    @pl.when(i == 0)
    def _init():
        s0_ref[0, 0] = 0.0
        s1_ref[0, 0] = 0.0

    f32 = jnp.float32

    def vq_idx(e, cbt_ref):
        cbt = cbt_ref[...]
        g = jax.lax.dot_general(e, cbt, (((1,), (0,)), ((), ())),
                                preferred_element_type=f32)
        sc = jnp.sum(cbt * cbt, axis=0, keepdims=True) - 2.0 * g
        m = jnp.min(sc, axis=1, keepdims=True)
        iota = jax.lax.broadcasted_iota(jnp.int32, sc.shape, 1)
        idx = jnp.min(jnp.where(sc == m, iota, _K), axis=1, keepdims=True)
        return idx, jnp.sum(e * e) + jnp.sum(m)

    x_all = jnp.concatenate([x0_ref[j] for j in range(4)], axis=0)
    e_all = jax.lax.dot_general(x_all, we0_ref[...], (((1,), (0,)), ((), ())),
                                preferred_element_type=f32) + be0_ref[...]
    idx0, s0 = vq_idx(e_all, cb0t_ref)
    acc_sc[...] = jnp.broadcast_to(be1_ref[...], (_BM1, _D))

    for j in range(4):
        ej = e_all[j * _BM1:(j + 1) * _BM1]
        enc0_ref[j] = ej
        idx0_ref[j] = idx0[j * _BM1:(j + 1) * _BM1]
        acc_sc[...] += jax.lax.dot_general(ej, we1_ref[j],
                                           (((1,), (0,)), ((), ())),
                                           preferred_element_type=f32)
    idx1, ds1 = vq_idx(acc_sc[...], cb1t_ref)
    idx1_ref[...] = idx1
    s0_ref[0, 0] += s0
    s1_ref[0, 0] += ds1


def _dec_body(x0_ref, enc0_ref, emb0_ref, emb1_ref, wd0_ref, bd0_ref,
              wup_ref, bup_ref, wout_ref, bout_ref, sm0_ref, sfin_ref):
    i = pl.program_id(0)

    @pl.when(i == 0)
    def _init():
        sm0_ref[0, 0] = 0.0
        sfin_ref[0, 0] = 0.0

    f32 = jnp.float32
    emb1 = emb1_ref[...]
    d0_all = jax.lax.dot_general(emb1, wd0_ref[...], (((1,), (0,)), ((), ())),
                                 preferred_element_type=f32) + bd0_ref[...]
    up_all = jax.lax.dot_general(emb1, wup_ref[...], (((1,), (0,)), ((), ())),
                                 preferred_element_type=f32) + bup_ref[...]

    sm0 = 0.0
    sfin = 0.0
    for j in range(4):
        d = d0_all[:, _D * j:_D * (j + 1)] - enc0_ref[j]
        sm0 = sm0 + jnp.sum(d * d)
        h = jnp.maximum(
            jnp.concatenate([up_all[:, _D * j:_D * (j + 1)], emb0_ref[j]],
                            axis=1), 0.0)
        r = jax.lax.dot_general(h, wout_ref[...], (((1,), (0,)), ((), ())),
                                preferred_element_type=f32) + bout_ref[...]
        dr = r - x0_ref[j]
        sfin = sfin + jnp.sum(dr * dr)

    sm0_ref[0, 0] += sm0
    sfin_ref[0, 0] += sfin


def _full(shape):
    return pl.BlockSpec(shape, lambda i: tuple(0 for _ in shape))


_SCALAR = pl.BlockSpec((1, 1), lambda i: (0, 0), memory_space=pltpu.SMEM)


def _enc_call(x0, we0, be0, we1, be1, cb0t, cb1t, interpret=False):
    return pl.pallas_call(
        _enc_body,
        grid=(_GRID,),
        in_specs=[
            pl.BlockSpec((4, _BM1, 48), lambda i: (0, i, 0)),
            _full((48, _D)), _full((1, _D)),
            _full((4, _D, _D)), _full((1, _D)),
            _full((_D, _K)), _full((_D, _K)),
        ],
        out_specs=[
            pl.BlockSpec((4, _BM1, _D), lambda i: (0, i, 0)),
            pl.BlockSpec((4, _BM1, 1), lambda i: (0, i, 0)),
            pl.BlockSpec((_BM1, 1), lambda i: (i, 0)),
            _SCALAR, _SCALAR,
        ],
        out_shape=[
            jax.ShapeDtypeStruct((4, _M1, _D), jnp.float32),
            jax.ShapeDtypeStruct((4, _M1, 1), jnp.int32),
            jax.ShapeDtypeStruct((_M1, 1), jnp.int32),
            jax.ShapeDtypeStruct((1, 1), jnp.float32),
            jax.ShapeDtypeStruct((1, 1), jnp.float32),
        ],
        scratch_shapes=[pltpu.VMEM((_BM1, _D), jnp.float32)],
        interpret=interpret,
    )(x0, we0, be0, we1, be1, cb0t, cb1t)


def _dec_call(x0, enc0, emb0, emb1, wd0, bd0, wup, bup, wout, bout,
              interpret=False):
    return pl.pallas_call(
        _dec_body,
        grid=(_GRID,),
        in_specs=[
            pl.BlockSpec((4, _BM1, 48), lambda i: (0, i, 0)),
            pl.BlockSpec((4, _BM1, _D), lambda i: (0, i, 0)),
            pl.BlockSpec((4, _BM1, _D), lambda i: (0, i, 0)),
            pl.BlockSpec((_BM1, _D), lambda i: (i, 0)),
            _full((_D, 4 * _D)), _full((1, 4 * _D)),
            _full((_D, 4 * _D)), _full((1, 4 * _D)),
            _full((2 * _D, 48)), _full((1, 48)),
        ],
        out_specs=[_SCALAR, _SCALAR],
        out_shape=[jax.ShapeDtypeStruct((1, 1), jnp.float32)] * 2,
        interpret=interpret,
    )(x0, enc0, emb0, emb1, wd0, bd0, wup, bup, wout, bout)


@functools.cache
def _make_gather(B):
    from jax.experimental.pallas import tpu_sc as plsc
    b_per_w = B // _NW
    mesh = plsc.VectorSubcoreMesh(core_axis_name="c", subcore_axis_name="s")

    @functools.partial(
        pl.kernel, mesh=mesh,
        out_type=jax.ShapeDtypeStruct((B, _D), jnp.float32),
        scratch_types=[pltpu.VMEM((b_per_w,), jnp.int32),
                       pltpu.VMEM((b_per_w, _D), jnp.float32),
                       pltpu.SemaphoreType.DMA],
    )
    def gather_k(table_hbm, idx_hbm, out_hbm, idx_v, rows_v, sem):
        wid = lax.axis_index("s") * 2 + lax.axis_index("c")
        base = wid * b_per_w
        pltpu.sync_copy(idx_hbm.at[pl.ds(base, b_per_w)], idx_v)
        pltpu.async_copy(table_hbm.at[idx_v], rows_v, sem).wait()
        pltpu.sync_copy(rows_v, out_hbm.at[pl.ds(base, b_per_w)])

    return gather_k


def kernel(inputs, W_enc0, b_enc0, W_enc1, b_enc1, codebook0, codebook1,
           W_dec0, b_dec0, W_dec1_up, b_dec1_up, W_dec1_out, b_dec1_out,
           commitment=0.25):
    x0 = (inputs.reshape(8, 28, 2, 4, 28, 2, 4, 3)
          .transpose(2, 5, 0, 1, 4, 3, 6, 7)
          .reshape(4, _M1, 48))
    we1 = W_enc1.reshape(4, _D, _D)

    enc0, idx0, idx1, s0, s1 = _enc_call(
        x0, W_enc0, b_enc0.reshape(1, _D), we1, b_enc1.reshape(1, _D),
        codebook0.T, codebook1.T)

    emb0 = _make_gather(_N0)(codebook0, idx0.reshape(_N0))
    idx1p = jnp.pad(idx1.reshape(_M1), (0, _N1PAD - _M1))
    emb1 = _make_gather(_N1PAD)(codebook1, idx1p)

    sm0, sfin = _dec_call(
        x0, enc0, emb0.reshape(4, _M1, _D), emb1,
        W_dec0, b_dec0.reshape(1, 4 * _D), W_dec1_up,
        b_dec1_up.reshape(1, 4 * _D), W_dec1_out, b_dec1_out.reshape(1, 48))

    vq_loss = (1.0 + commitment) * (s0[0, 0] / (_N0 * _D)
                                    + s1[0, 0] / (_M1 * _D))
    mse0 = sm0[0, 0] / (_M1 * 4 * _D)
    final_mse = sfin[0, 0] / (_N0 * 48.0)
    loss = vq_loss + mse0 + final_mse
    return loss, final_mse


# speedup vs baseline: 1.8440x; 1.0300x over previous
import functools

import jax
import jax.numpy as jnp
from jax import lax
from jax.experimental import pallas as pl
from jax.experimental.pallas import tpu as pltpu

_M1 = 6272
_BM1 = 784
_GRID = _M1 // _BM1
_K = 512
_D = 128
_N0 = 4 * _M1
_N1PAD = 6400
_NW = 32


def _enc_body(x0_ref, we0_ref, be0_ref, we1_ref, be1_ref, cb0t_ref, cb1t_ref,
              enc0_ref, idx0_ref, idx1_ref, s0_ref, s1_ref, acc_sc):
    i = pl.program_id(0)

    @pl.when(i == 0)
    def _init():
        s0_ref[0, 0] = 0.0
        s1_ref[0, 0] = 0.0

    f32 = jnp.float32

    def vq_idx(e, cbt_ref):
        cbt = cbt_ref[...]
        g = jax.lax.dot_general(e, cbt, (((1,), (0,)), ((), ())),
                                preferred_element_type=f32)
        sc = jnp.sum(cbt * cbt, axis=0, keepdims=True) - 2.0 * g
        m = jnp.min(sc, axis=1, keepdims=True)
        iota = jax.lax.broadcasted_iota(jnp.int32, sc.shape, 1)
        idx = jnp.min(jnp.where(sc == m, iota, _K), axis=1, keepdims=True)
        return idx, jnp.sum(e * e) + jnp.sum(m)

    x_all = jnp.concatenate([x0_ref[j] for j in range(4)], axis=0)
    e_all = jax.lax.dot_general(x_all, we0_ref[...], (((1,), (0,)), ((), ())),
                                preferred_element_type=f32) + be0_ref[...]
    idx0, s0 = vq_idx(e_all, cb0t_ref)
    acc_sc[...] = jnp.broadcast_to(be1_ref[...], (_BM1, _D))

    for j in range(4):
        ej = e_all[j * _BM1:(j + 1) * _BM1]
        enc0_ref[j] = ej
        idx0_ref[j] = idx0[j * _BM1:(j + 1) * _BM1]
        acc_sc[...] += jax.lax.dot_general(ej, we1_ref[j],
                                           (((1,), (0,)), ((), ())),
                                           preferred_element_type=f32)
    idx1, ds1 = vq_idx(acc_sc[...], cb1t_ref)
    idx1_ref[...] = idx1
    s0_ref[0, 0] += s0
    s1_ref[0, 0] += ds1


def _dec_body(x0_ref, enc0_ref, emb0_ref, emb1_ref, wd0_ref, bd0_ref,
              wup_ref, bup_ref, wout_ref, bout_ref, sm0_ref, sfin_ref):
    i = pl.program_id(0)

    @pl.when(i == 0)
    def _init():
        sm0_ref[0, 0] = 0.0
        sfin_ref[0, 0] = 0.0

    f32 = jnp.float32
    emb1 = emb1_ref[...]
    d0_all = jax.lax.dot_general(emb1, wd0_ref[...], (((1,), (0,)), ((), ())),
                                 preferred_element_type=f32) + bd0_ref[...]
    up_all = jax.lax.dot_general(emb1, wup_ref[...], (((1,), (0,)), ((), ())),
                                 preferred_element_type=f32) + bup_ref[...]

    sm0 = 0.0
    sfin = 0.0
    for j in range(4):
        d = d0_all[:, _D * j:_D * (j + 1)] - enc0_ref[j]
        sm0 = sm0 + jnp.sum(d * d)
        h = jnp.maximum(
            jnp.concatenate([up_all[:, _D * j:_D * (j + 1)], emb0_ref[j]],
                            axis=1), 0.0)
        r = jax.lax.dot_general(h, wout_ref[...], (((1,), (0,)), ((), ())),
                                preferred_element_type=f32) + bout_ref[...]
        dr = r - x0_ref[j]
        sfin = sfin + jnp.sum(dr * dr)

    sm0_ref[0, 0] += sm0
    sfin_ref[0, 0] += sfin


def _full(shape):
    return pl.BlockSpec(shape, lambda i: tuple(0 for _ in shape))


_SCALAR = pl.BlockSpec((1, 1), lambda i: (0, 0), memory_space=pltpu.SMEM)


def _enc_call(x0, we0, be0, we1, be1, cb0t, cb1t, interpret=False):
    return pl.pallas_call(
        _enc_body,
        grid=(_GRID,),
        in_specs=[
            pl.BlockSpec((4, _BM1, 48), lambda i: (0, i, 0)),
            _full((48, _D)), _full((1, _D)),
            _full((4, _D, _D)), _full((1, _D)),
            _full((_D, _K)), _full((_D, _K)),
        ],
        out_specs=[
            pl.BlockSpec((4, _BM1, _D), lambda i: (0, i, 0)),
            pl.BlockSpec((4, _BM1, 1), lambda i: (0, i, 0)),
            pl.BlockSpec((_BM1, 1), lambda i: (i, 0)),
            _SCALAR, _SCALAR,
        ],
        out_shape=[
            jax.ShapeDtypeStruct((4, _M1, _D), jnp.float32),
            jax.ShapeDtypeStruct((4, _M1, 1), jnp.int32),
            jax.ShapeDtypeStruct((_M1, 1), jnp.int32),
            jax.ShapeDtypeStruct((1, 1), jnp.float32),
            jax.ShapeDtypeStruct((1, 1), jnp.float32),
        ],
        scratch_shapes=[pltpu.VMEM((_BM1, _D), jnp.float32)],
        interpret=interpret,
    )(x0, we0, be0, we1, be1, cb0t, cb1t)


def _dec_call(x0, enc0, emb0, emb1, wd0, bd0, wup, bup, wout, bout,
              interpret=False):
    return pl.pallas_call(
        _dec_body,
        grid=(_GRID,),
        in_specs=[
            pl.BlockSpec((4, _BM1, 48), lambda i: (0, i, 0)),
            pl.BlockSpec((4, _BM1, _D), lambda i: (0, i, 0)),
            pl.BlockSpec((4, _BM1, _D), lambda i: (0, i, 0)),
            pl.BlockSpec((_BM1, _D), lambda i: (i, 0)),
            _full((_D, 4 * _D)), _full((1, 4 * _D)),
            _full((_D, 4 * _D)), _full((1, 4 * _D)),
            _full((2 * _D, 48)), _full((1, 48)),
        ],
        out_specs=[_SCALAR, _SCALAR],
        out_shape=[jax.ShapeDtypeStruct((1, 1), jnp.float32)] * 2,
        interpret=interpret,
    )(x0, enc0, emb0, emb1, wd0, bd0, wup, bup, wout, bout)


@functools.cache
def _make_gather(B):
    from jax.experimental.pallas import tpu_sc as plsc
    b_per_w = B // _NW
    mesh = plsc.VectorSubcoreMesh(core_axis_name="c", subcore_axis_name="s")

    @functools.partial(
        pl.kernel, mesh=mesh,
        out_type=jax.ShapeDtypeStruct((B, _D), jnp.float32),
        scratch_types=[pltpu.VMEM((b_per_w,), jnp.int32),
                       pltpu.VMEM((b_per_w, _D), jnp.float32),
                       pltpu.SemaphoreType.DMA],
    )
    def gather_k(table_hbm, idx_hbm, out_hbm, idx_v, rows_v, sem):
        wid = lax.axis_index("s") * 2 + lax.axis_index("c")
        base = wid * b_per_w
        pltpu.sync_copy(idx_hbm.at[pl.ds(base, b_per_w)], idx_v)
        pltpu.async_copy(table_hbm.at[idx_v], rows_v, sem).wait()
        pltpu.sync_copy(rows_v, out_hbm.at[pl.ds(base, b_per_w)])

    return gather_k


def kernel(inputs, W_enc0, b_enc0, W_enc1, b_enc1, codebook0, codebook1,
           W_dec0, b_dec0, W_dec1_up, b_dec1_up, W_dec1_out, b_dec1_out,
           commitment=0.25):
    x0 = (inputs.reshape(8, 28, 2, 4, 28, 2, 4, 3)
          .transpose(2, 5, 0, 1, 4, 3, 6, 7)
          .reshape(4, _M1, 48))
    we1 = W_enc1.reshape(4, _D, _D)

    enc0, idx0, idx1, s0, s1 = _enc_call(
        x0, W_enc0, b_enc0.reshape(1, _D), we1, b_enc1.reshape(1, _D),
        codebook0.T, codebook1.T)

    emb0 = _make_gather(_N0)(codebook0, idx0.reshape(_N0))
    idx1p = jnp.pad(idx1.reshape(_M1), (0, _N1PAD - _M1))
    emb1 = _make_gather(_N1PAD)(codebook1, idx1p)

    sm0, sfin = _dec_call(
        x0, enc0, emb0.reshape(4, _M1, _D), emb1,
        W_dec0, b_dec0.reshape(1, 4 * _D), W_dec1_up,
        b_dec1_up.reshape(1, 4 * _D), W_dec1_out, b_dec1_out.reshape(1, 48))

    vq_loss = (1.0 + commitment) * (s0[0, 0] / (_N0 * _D)
                                    + s1[0, 0] / (_M1 * _D))
    mse0 = sm0[0, 0] / (_M1 * 4 * _D)
    final_mse = sfin[0, 0] / (_N0 * 48.0)
    loss = vq_loss + mse0 + final_mse
    return loss, final_mse


# speedup vs baseline: 1.9274x; 1.0453x over previous
import functools

import jax
import jax.numpy as jnp
from jax import lax
from jax.experimental import pallas as pl
from jax.experimental.pallas import tpu as pltpu

_M1 = 6272
_BM1 = 784
_GRID = _M1 // _BM1
_K = 512
_D = 128
_N0 = 4 * _M1
_N1PAD = 6400
_NW = 32


def _enc_body(x0_ref, we0_ref, be0_ref, we1_ref, be1_ref, cb0t_ref, cb1t_ref,
              idx0_ref, idx1_ref, s0_ref, s1_ref, acc_sc):
    i = pl.program_id(0)

    @pl.when(i == 0)
    def _init():
        s0_ref[0, 0] = 0.0
        s1_ref[0, 0] = 0.0

    f32 = jnp.float32

    def vq_idx(e, cbt_ref):
        cbt = cbt_ref[...]
        g = jax.lax.dot_general(e, cbt, (((1,), (0,)), ((), ())),
                                preferred_element_type=f32)
        sc = jnp.sum(cbt * cbt, axis=0, keepdims=True) - 2.0 * g
        m = jnp.min(sc, axis=1, keepdims=True)
        iota = jax.lax.broadcasted_iota(jnp.int32, sc.shape, 1)
        idx = jnp.min(jnp.where(sc == m, iota, _K), axis=1, keepdims=True)
        return idx, jnp.sum(e * e) + jnp.sum(m)

    x_all = jnp.concatenate([x0_ref[j] for j in range(4)], axis=0)
    e_all = jax.lax.dot_general(x_all, we0_ref[...], (((1,), (0,)), ((), ())),
                                preferred_element_type=f32) + be0_ref[...]
    idx0, s0 = vq_idx(e_all, cb0t_ref)
    acc_sc[...] = jnp.broadcast_to(be1_ref[...], (_BM1, _D))

    for j in range(4):
        ej = e_all[j * _BM1:(j + 1) * _BM1]
        idx0_ref[j] = idx0[j * _BM1:(j + 1) * _BM1]
        acc_sc[...] += jax.lax.dot_general(ej, we1_ref[j],
                                           (((1,), (0,)), ((), ())),
                                           preferred_element_type=f32)
    idx1, ds1 = vq_idx(acc_sc[...], cb1t_ref)
    idx1_ref[...] = idx1
    s0_ref[0, 0] += s0
    s1_ref[0, 0] += ds1


def _dec_body(x0_ref, we0_ref, be0_ref, emb0_ref, emb1_ref, wd0_ref, bd0_ref,
              wup_ref, bup_ref, wout_ref, bout_ref, sm0_ref, sfin_ref):
    i = pl.program_id(0)

    @pl.when(i == 0)
    def _init():
        sm0_ref[0, 0] = 0.0
        sfin_ref[0, 0] = 0.0

    f32 = jnp.float32
    x_all = jnp.concatenate([x0_ref[j] for j in range(4)], axis=0)
    e_all = jax.lax.dot_general(x_all, we0_ref[...], (((1,), (0,)), ((), ())),
                                preferred_element_type=f32) + be0_ref[...]
    emb1 = emb1_ref[...]
    d0_all = jax.lax.dot_general(emb1, wd0_ref[...], (((1,), (0,)), ((), ())),
                                 preferred_element_type=f32) + bd0_ref[...]
    up_all = jax.lax.dot_general(emb1, wup_ref[...], (((1,), (0,)), ((), ())),
                                 preferred_element_type=f32) + bup_ref[...]

    sm0 = 0.0
    sfin = 0.0
    for j in range(4):
        d = d0_all[:, _D * j:_D * (j + 1)] - e_all[j * _BM1:(j + 1) * _BM1]
        sm0 = sm0 + jnp.sum(d * d)
        h = jnp.maximum(
            jnp.concatenate([up_all[:, _D * j:_D * (j + 1)], emb0_ref[j]],
                            axis=1), 0.0)
        r = jax.lax.dot_general(h, wout_ref[...], (((1,), (0,)), ((), ())),
                                preferred_element_type=f32) + bout_ref[...]
        dr = r - x0_ref[j]
        sfin = sfin + jnp.sum(dr * dr)

    sm0_ref[0, 0] += sm0
    sfin_ref[0, 0] += sfin


def _full(shape):
    return pl.BlockSpec(shape, lambda i: tuple(0 for _ in shape))


_SCALAR = pl.BlockSpec((1, 1), lambda i: (0, 0), memory_space=pltpu.SMEM)


def _enc_call(x0, we0, be0, we1, be1, cb0t, cb1t, interpret=False):
    return pl.pallas_call(
        _enc_body,
        grid=(_GRID,),
        in_specs=[
            pl.BlockSpec((4, _BM1, 48), lambda i: (0, i, 0)),
            _full((48, _D)), _full((1, _D)),
            _full((4, _D, _D)), _full((1, _D)),
            _full((_D, _K)), _full((_D, _K)),
        ],
        out_specs=[
            pl.BlockSpec((4, _BM1, 1), lambda i: (0, i, 0)),
            pl.BlockSpec((_BM1, 1), lambda i: (i, 0)),
            _SCALAR, _SCALAR,
        ],
        out_shape=[
            jax.ShapeDtypeStruct((4, _M1, 1), jnp.int32),
            jax.ShapeDtypeStruct((_M1, 1), jnp.int32),
            jax.ShapeDtypeStruct((1, 1), jnp.float32),
            jax.ShapeDtypeStruct((1, 1), jnp.float32),
        ],
        scratch_shapes=[pltpu.VMEM((_BM1, _D), jnp.float32)],
        interpret=interpret,
    )(x0, we0, be0, we1, be1, cb0t, cb1t)


def _dec_call(x0, we0, be0, emb0, emb1, wd0, bd0, wup, bup, wout, bout,
              interpret=False):
    return pl.pallas_call(
        _dec_body,
        grid=(_GRID,),
        in_specs=[
            pl.BlockSpec((4, _BM1, 48), lambda i: (0, i, 0)),
            _full((48, _D)), _full((1, _D)),
            pl.BlockSpec((4, _BM1, _D), lambda i: (0, i, 0)),
            pl.BlockSpec((_BM1, _D), lambda i: (i, 0)),
            _full((_D, 4 * _D)), _full((1, 4 * _D)),
            _full((_D, 4 * _D)), _full((1, 4 * _D)),
            _full((2 * _D, 48)), _full((1, 48)),
        ],
        out_specs=[_SCALAR, _SCALAR],
        out_shape=[jax.ShapeDtypeStruct((1, 1), jnp.float32)] * 2,
        interpret=interpret,
    )(x0, we0, be0, emb0, emb1, wd0, bd0, wup, bup, wout, bout)


@functools.cache
def _make_gather(B):
    from jax.experimental.pallas import tpu_sc as plsc
    b_per_w = B // _NW
    mesh = plsc.VectorSubcoreMesh(core_axis_name="c", subcore_axis_name="s")

    @functools.partial(
        pl.kernel, mesh=mesh,
        out_type=jax.ShapeDtypeStruct((B, _D), jnp.float32),
        scratch_types=[pltpu.VMEM((b_per_w,), jnp.int32),
                       pltpu.VMEM((b_per_w, _D), jnp.float32),
                       pltpu.SemaphoreType.DMA],
    )
    def gather_k(table_hbm, idx_hbm, out_hbm, idx_v, rows_v, sem):
        wid = lax.axis_index("s") * 2 + lax.axis_index("c")
        base = wid * b_per_w
        pltpu.sync_copy(idx_hbm.at[pl.ds(base, b_per_w)], idx_v)
        pltpu.async_copy(table_hbm.at[idx_v], rows_v, sem).wait()
        pltpu.sync_copy(rows_v, out_hbm.at[pl.ds(base, b_per_w)])

    return gather_k


def kernel(inputs, W_enc0, b_enc0, W_enc1, b_enc1, codebook0, codebook1,
           W_dec0, b_dec0, W_dec1_up, b_dec1_up, W_dec1_out, b_dec1_out,
           commitment=0.25):
    x0 = (inputs.reshape(8, 28, 2, 4, 28, 2, 4, 3)
          .transpose(2, 5, 0, 1, 4, 3, 6, 7)
          .reshape(4, _M1, 48))
    we1 = W_enc1.reshape(4, _D, _D)

    idx0, idx1, s0, s1 = _enc_call(
        x0, W_enc0, b_enc0.reshape(1, _D), we1, b_enc1.reshape(1, _D),
        codebook0.T, codebook1.T)

    emb0 = _make_gather(_N0)(codebook0, idx0.reshape(_N0))
    idx1p = jnp.pad(idx1.reshape(_M1), (0, _N1PAD - _M1))
    emb1 = _make_gather(_N1PAD)(codebook1, idx1p)

    sm0, sfin = _dec_call(
        x0, W_enc0, b_enc0.reshape(1, _D), emb0.reshape(4, _M1, _D), emb1,
        W_dec0, b_dec0.reshape(1, 4 * _D), W_dec1_up,
        b_dec1_up.reshape(1, 4 * _D), W_dec1_out, b_dec1_out.reshape(1, 48))

    vq_loss = (1.0 + commitment) * (s0[0, 0] / (_N0 * _D)
                                    + s1[0, 0] / (_M1 * _D))
    mse0 = sm0[0, 0] / (_M1 * 4 * _D)
    final_mse = sfin[0, 0] / (_N0 * 48.0)
    loss = vq_loss + mse0 + final_mse
    return loss, final_mse
